# baseline (speedup 1.0000x reference)
import jax
import jax.numpy as jnp
from jax import lax
from jax.experimental import pallas as pl

_BB = 32


def _loss_block(x_ref, cls_ref, pb_ref, tb_ref,
                ce_acc, m_acc, l1_acc, gi_acc, out_ref):
    b = pl.program_id(0)
    nb = pl.num_programs(0)
    c = x_ref.shape[0]

    @pl.when(b == 0)
    def _init():
        ce_acc[...] = jnp.zeros_like(ce_acc)
        m_acc[...] = jnp.zeros_like(m_acc)
        l1_acc[...] = jnp.zeros_like(l1_acc)
        gi_acc[...] = jnp.zeros_like(gi_acc)

    cls = cls_ref[...]

    def focal_chunk(base, cc):
        x = x_ref[pl.ds(base, cc), :, :]
        cidx = base + lax.broadcasted_iota(jnp.int32, x.shape, 0)
        t = cidx == cls[None, :, :]
        y = jnp.where(t, -x, x)
        e = jnp.exp(-jnp.abs(x))
        lp = jnp.log1p(e)
        s = 1.0 / (1.0 + e)
        sp = jnp.maximum(y, 0.0) + lp
        sg = jnp.where(y >= 0.0, s, 1.0 - s)
        alpha = jnp.where(t, 0.25, 0.75)
        loss = alpha * sp * sg * sg
        return jnp.sum(loss, axis=0)

    cc = 8
    part = focal_chunk(0, cc)
    for i in range(1, c // cc):
        part = part + focal_chunk(i * cc, cc)
    if c % cc:
        part = part + focal_chunk(c - c % cc, c % cc)
    ce_acc[...] += part

    matched = (cls != c).astype(jnp.float32)
    pb = pb_ref[...]
    tbx = tb_ref[...]
    l1 = jnp.sum(jnp.abs(pb - tbx), axis=1) * matched

    def corners(bx):
        cx = bx[:, 0, :]
        cy = bx[:, 1, :]
        w = bx[:, 2, :]
        h = bx[:, 3, :]
        return cx - 0.5 * w, cy - 0.5 * h, cx + 0.5 * w, cy + 0.5 * h

    ax0, ay0, ax1, ay1 = corners(pb)
    bx0, by0, bx1, by1 = corners(tbx)
    area_a = (ax1 - ax0) * (ay1 - ay0)
    area_b = (bx1 - bx0) * (by1 - by0)
    iw = jnp.maximum(jnp.minimum(ax1, bx1) - jnp.maximum(ax0, bx0), 0.0)
    ih = jnp.maximum(jnp.minimum(ay1, by1) - jnp.maximum(ay0, by0), 0.0)
    inter = iw * ih
    union = area_a + area_b - inter
    iou = inter / (union + 1e-7)
    ew = jnp.maximum(jnp.maximum(ax1, bx1) - jnp.minimum(ax0, bx0), 0.0)
    eh = jnp.maximum(jnp.maximum(ay1, by1) - jnp.minimum(ay0, by0), 0.0)
    area_e = ew * eh
    giou = iou - (area_e - union) / (area_e + 1e-7)

    m_acc[...] += matched
    l1_acc[...] += l1
    gi_acc[...] += (1.0 - giou) * matched

    @pl.when(b == nb - 1)
    def _final():
        s_ce = jnp.sum(ce_acc[...])
        s_m = jnp.sum(m_acc[...])
        s_l1 = jnp.sum(l1_acc[...])
        s_gi = jnp.sum(gi_acc[...])
        num_boxes = jnp.maximum(s_m, 1.0)
        lane3 = lax.broadcasted_iota(jnp.int32, (1, 3), 1)
        out_ref[...] = (s_ce * (lane3 == 0) + s_l1 * (lane3 == 1)
                        + s_gi * (lane3 == 2)) / num_boxes


def kernel(logits, pred_boxes, target_boxes, target_classes):
    B, Q, C = logits.shape
    xt = jnp.transpose(logits, (2, 0, 1))
    pbt = jnp.transpose(pred_boxes, (0, 2, 1))
    tbt = jnp.transpose(target_boxes, (0, 2, 1))
    cls = target_classes.astype(jnp.int32)
    grid = B // _BB
    outs = pl.pallas_call(
        _loss_block,
        grid=(grid,),
        in_specs=[
            pl.BlockSpec((C, _BB, Q), lambda b: (0, b, 0)),
            pl.BlockSpec((_BB, Q), lambda b: (b, 0)),
            pl.BlockSpec((_BB, 4, Q), lambda b: (b, 0, 0)),
            pl.BlockSpec((_BB, 4, Q), lambda b: (b, 0, 0)),
        ],
        out_specs=[
            pl.BlockSpec((_BB, Q), lambda b: (0, 0)),
            pl.BlockSpec((_BB, Q), lambda b: (0, 0)),
            pl.BlockSpec((_BB, Q), lambda b: (0, 0)),
            pl.BlockSpec((_BB, Q), lambda b: (0, 0)),
            pl.BlockSpec((1, 3), lambda b: (0, 0)),
        ],
        out_shape=[
            jax.ShapeDtypeStruct((_BB, Q), jnp.float32),
            jax.ShapeDtypeStruct((_BB, Q), jnp.float32),
            jax.ShapeDtypeStruct((_BB, Q), jnp.float32),
            jax.ShapeDtypeStruct((_BB, Q), jnp.float32),
            jax.ShapeDtypeStruct((1, 3), jnp.float32),
        ],
    )(xt, cls, pbt, tbt)
    return outs[4][0]


# speedup vs baseline: 1.0880x; 1.0880x over previous
import jax
import jax.numpy as jnp
from jax import lax
from jax.experimental import pallas as pl

_BB = 16


def _loss_block(x_ref, cls_ref, pb_ref, tb_ref,
                ce_acc, m_acc, l1_acc, gi_acc, out_ref):
    b = pl.program_id(0)
    nb = pl.num_programs(0)
    c = x_ref.shape[0]

    @pl.when(b == 0)
    def _init():
        ce_acc[...] = jnp.zeros_like(ce_acc)
        m_acc[...] = jnp.zeros_like(m_acc)
        l1_acc[...] = jnp.zeros_like(l1_acc)
        gi_acc[...] = jnp.zeros_like(gi_acc)

    cls = cls_ref[...]

    def focal_chunk(base, cc):
        x = x_ref[pl.ds(base, cc), :, :]
        cidx = base + lax.broadcasted_iota(jnp.int32, x.shape, 0)
        t = cidx == cls[None, :, :]
        nx = -x
        y = jnp.where(t, nx, x)
        e = jnp.exp(jnp.minimum(x, nx))
        lp = jnp.log1p(e)
        s = 1.0 / (1.0 + e)
        sp = jnp.maximum(y, 0.0) + lp
        sg = jnp.where(y >= 0.0, s, 1.0 - s)
        alpha = jnp.where(t, 0.25, 0.75)
        loss = alpha * sp * sg * sg
        return jnp.sum(loss, axis=0)

    cc = 8
    part = focal_chunk(0, cc)
    for i in range(1, c // cc):
        part = part + focal_chunk(i * cc, cc)
    if c % cc:
        part = part + focal_chunk(c - c % cc, c % cc)
    ce_acc[...] += part

    matched = (cls != c).astype(jnp.float32)
    pb = pb_ref[...]
    tbx = tb_ref[...]
    l1 = jnp.sum(jnp.abs(pb - tbx), axis=1) * matched

    def corners(bx):
        cx = bx[:, 0, :]
        cy = bx[:, 1, :]
        w = bx[:, 2, :]
        h = bx[:, 3, :]
        return cx - 0.5 * w, cy - 0.5 * h, cx + 0.5 * w, cy + 0.5 * h

    ax0, ay0, ax1, ay1 = corners(pb)
    bx0, by0, bx1, by1 = corners(tbx)
    area_a = (ax1 - ax0) * (ay1 - ay0)
    area_b = (bx1 - bx0) * (by1 - by0)
    iw = jnp.maximum(jnp.minimum(ax1, bx1) - jnp.maximum(ax0, bx0), 0.0)
    ih = jnp.maximum(jnp.minimum(ay1, by1) - jnp.maximum(ay0, by0), 0.0)
    inter = iw * ih
    union = area_a + area_b - inter
    iou = inter / (union + 1e-7)
    ew = jnp.maximum(jnp.maximum(ax1, bx1) - jnp.minimum(ax0, bx0), 0.0)
    eh = jnp.maximum(jnp.maximum(ay1, by1) - jnp.minimum(ay0, by0), 0.0)
    area_e = ew * eh
    giou = iou - (area_e - union) / (area_e + 1e-7)

    m_acc[...] += matched
    l1_acc[...] += l1
    gi_acc[...] += (1.0 - giou) * matched

    @pl.when(b == nb - 1)
    def _final():
        s_ce = jnp.sum(ce_acc[...])
        s_m = jnp.sum(m_acc[...])
        s_l1 = jnp.sum(l1_acc[...])
        s_gi = jnp.sum(gi_acc[...])
        num_boxes = jnp.maximum(s_m, 1.0)
        lane3 = lax.broadcasted_iota(jnp.int32, (1, 3), 1)
        out_ref[...] = (s_ce * (lane3 == 0) + s_l1 * (lane3 == 1)
                        + s_gi * (lane3 == 2)) / num_boxes


def kernel(logits, pred_boxes, target_boxes, target_classes):
    B, Q, C = logits.shape
    xt = jnp.transpose(logits, (2, 0, 1))
    pbt = jnp.transpose(pred_boxes, (0, 2, 1))
    tbt = jnp.transpose(target_boxes, (0, 2, 1))
    cls = target_classes.astype(jnp.int32)
    grid = B // _BB
    outs = pl.pallas_call(
        _loss_block,
        grid=(grid,),
        in_specs=[
            pl.BlockSpec((C, _BB, Q), lambda b: (0, b, 0)),
            pl.BlockSpec((_BB, Q), lambda b: (b, 0)),
            pl.BlockSpec((_BB, 4, Q), lambda b: (b, 0, 0)),
            pl.BlockSpec((_BB, 4, Q), lambda b: (b, 0, 0)),
        ],
        out_specs=[
            pl.BlockSpec((_BB, Q), lambda b: (0, 0)),
            pl.BlockSpec((_BB, Q), lambda b: (0, 0)),
            pl.BlockSpec((_BB, Q), lambda b: (0, 0)),
            pl.BlockSpec((_BB, Q), lambda b: (0, 0)),
            pl.BlockSpec((1, 3), lambda b: (0, 0)),
        ],
        out_shape=[
            jax.ShapeDtypeStruct((_BB, Q), jnp.float32),
            jax.ShapeDtypeStruct((_BB, Q), jnp.float32),
            jax.ShapeDtypeStruct((_BB, Q), jnp.float32),
            jax.ShapeDtypeStruct((_BB, Q), jnp.float32),
            jax.ShapeDtypeStruct((1, 3), jnp.float32),
        ],
    )(xt, cls, pbt, tbt)
    return outs[4][0]
